# baseline (speedup 1.0000x reference)
import functools
import math

import jax
import jax.numpy as jnp
from jax import lax
from jax.experimental import pallas as pl
from jax.experimental.pallas import tpu as pltpu
from jax.experimental.pallas import tpu_sc as plsc

D_MODEL = 1024
N_EXPERTS = 16
TOP_K = 2
CAP_FACTOR = 1.25
NUM_TOKENS = 2048


def _cap(num_tokens):
    cap = math.floor(TOP_K * CAP_FACTOR * num_tokens / N_EXPERTS)
    cap += cap % 2
    return max(int(cap), 2)


CAP = _cap(NUM_TOKENS)
SLOTS = N_EXPERTS * CAP
T_PAD = NUM_TOKENS + 8
TB = 256
NG = NUM_TOKENS // TB

SC_CORES = 2
SC_SUBCORES = 16
NW = SC_CORES * SC_SUBCORES
ROWS_PER_W = SLOTS // NW
ROW_CHUNK = 80


def _route_body(x_ref, w_ref, logits_ref, xpad_ref,
                slot0_ref, slot1_ref, p0_ref, p1_ref):
    T, E = NUM_TOKENS, N_EXPERTS
    x = x_ref[...]
    xpad_ref[0:T, :] = x
    xpad_ref[T:T_PAD, :] = jnp.zeros((T_PAD - T, D_MODEL), jnp.float32)
    logits = lax.dot_general(
        x, w_ref[...], (((1,), (1,)), ((), ())),
        preferred_element_type=jnp.float32,
        precision=lax.Precision.DEFAULT)
    logits_ref[...] = logits

    lane_e = lax.broadcasted_iota(jnp.int32, (T, E), 1)
    m0 = jnp.max(logits, axis=1, keepdims=True)
    i0 = jnp.min(jnp.where(logits == m0, lane_e, E), axis=1, keepdims=True)
    sel0 = lane_e == i0
    masked = jnp.where(sel0, -jnp.inf, logits)
    m1 = jnp.max(masked, axis=1, keepdims=True)
    i1 = jnp.min(jnp.where(masked == m1, lane_e, E), axis=1, keepdims=True)
    sel1 = lane_e == i1

    e1 = jnp.exp(m1 - m0)
    denom = 1.0 + e1
    p0 = 1.0 / denom
    p1 = e1 / denom

    oh0 = sel0.astype(jnp.float32)
    oh1 = sel1.astype(jnp.float32)
    r = lax.broadcasted_iota(jnp.int32, (T, T), 0)
    c = lax.broadcasted_iota(jnp.int32, (T, T), 1)
    trilf = (r >= c).astype(jnp.float32)
    cum0 = lax.dot_general(trilf, oh0, (((1,), (0,)), ((), ())),
                           preferred_element_type=jnp.float32)
    cum1 = lax.dot_general(trilf, oh1, (((1,), (0,)), ((), ())),
                           preferred_element_type=jnp.float32)
    rank0 = jnp.sum(jnp.where(sel0, cum0 - 1.0, 0.0), axis=1, keepdims=True)
    tot0 = jnp.sum(oh0, axis=0, keepdims=True)
    rank1 = jnp.sum(jnp.where(sel1, cum1 - 1.0 + tot0, 0.0),
                    axis=1, keepdims=True)

    capf = jnp.float32(CAP)
    slot0 = jnp.where(rank0 < capf,
                      i0.astype(jnp.float32) * capf + rank0,
                      jnp.float32(SLOTS))
    slot1 = jnp.where(rank1 < capf,
                      i1.astype(jnp.float32) * capf + rank1,
                      jnp.float32(SLOTS))
    slot0_ref[...] = slot0.astype(jnp.int32)
    slot1_ref[...] = slot1.astype(jnp.int32)
    p0_ref[...] = p0
    p1_ref[...] = p1


def _route(x2, w_gate):
    T = NUM_TOKENS
    return pl.pallas_call(
        _route_body,
        out_shape=[
            jax.ShapeDtypeStruct((T, N_EXPERTS), jnp.float32),
            jax.ShapeDtypeStruct((T_PAD, D_MODEL), jnp.float32),
            jax.ShapeDtypeStruct((T, 1), jnp.int32),
            jax.ShapeDtypeStruct((T, 1), jnp.int32),
            jax.ShapeDtypeStruct((T, 1), jnp.float32),
            jax.ShapeDtypeStruct((T, 1), jnp.float32),
        ],
    )(x2, w_gate)


def _build_body(slot0_ref, slot1_ref, p0_ref, p1_ref,
                w_out_ref, mask_ref, acc_ref, idx_ref):
    g = pl.program_id(0)
    s0 = slot0_ref[...]
    s1 = slot1_ref[...]
    lane = lax.broadcasted_iota(jnp.int32, (TB, SLOTS), 1)
    W = (jnp.where(lane == s0, p0_ref[...], 0.0)
         + jnp.where(lane == s1, p1_ref[...], 0.0))
    w_out_ref[...] = W
    nz = W != 0.0
    mask_ref[...] = nz

    tok = (g * TB + 1
           + lax.broadcasted_iota(jnp.int32, (TB, 1), 0))
    contrib = jnp.sum(jnp.where(nz, tok, 0), axis=0, keepdims=True)

    @pl.when(g == 0)
    def _():
        acc_ref[...] = contrib

    @pl.when(g > 0)
    def _():
        acc_ref[...] = acc_ref[...] + contrib

    @pl.when(g == NG - 1)
    def _():
        a = acc_ref[...]
        idx_ref[...] = jnp.where(a > 0, a - 1, NUM_TOKENS)


def _build(slot0, slot1, p0, p1):
    T = NUM_TOKENS
    col = pl.BlockSpec((TB, 1), lambda g: (g, 0))
    full = pl.BlockSpec((1, SLOTS), lambda g: (0, 0))
    return pl.pallas_call(
        _build_body,
        grid=(NG,),
        in_specs=[col, col, col, col],
        out_specs=[
            pl.BlockSpec((TB, SLOTS), lambda g: (g, 0)),
            pl.BlockSpec((TB, SLOTS), lambda g: (g, 0)),
            full,
            full,
        ],
        out_shape=[
            jax.ShapeDtypeStruct((T, SLOTS), jnp.float32),
            jax.ShapeDtypeStruct((T, SLOTS), jnp.bool_),
            jax.ShapeDtypeStruct((1, SLOTS), jnp.int32),
            jax.ShapeDtypeStruct((1, SLOTS), jnp.int32),
        ],
    )(slot0, slot1, p0, p1)


def _gather_body(xpad_hbm, idx_hbm, out_hbm, idx_v, rows_v, sem):
    wid = lax.axis_index("s") * SC_CORES + lax.axis_index("c")
    base = wid * ROWS_PER_W
    for ch in range(ROWS_PER_W // ROW_CHUNK):
        off = base + ch * ROW_CHUNK
        pltpu.sync_copy(idx_hbm.at[pl.ds(off, ROW_CHUNK)], idx_v)
        pltpu.async_copy(xpad_hbm.at[idx_v], rows_v, sem).wait()
        pltpu.sync_copy(rows_v, out_hbm.at[pl.ds(off, ROW_CHUNK)])


@functools.cache
def _make_gather():
    return pl.kernel(
        _gather_body,
        mesh=plsc.VectorSubcoreMesh(core_axis_name="c", subcore_axis_name="s",
                                    num_cores=SC_CORES),
        out_type=jax.ShapeDtypeStruct((SLOTS, D_MODEL), jnp.float32),
        scratch_types=[
            pltpu.VMEM((ROW_CHUNK,), jnp.int32),
            pltpu.VMEM((ROW_CHUNK, D_MODEL), jnp.float32),
            pltpu.SemaphoreType.DMA,
        ],
    )


@jax.jit
def kernel(x, w_gate):
    b, s, d = x.shape
    x2 = x.reshape(b * s, d)
    logits, xpad, slot0, slot1, p0, p1 = _route(x2, w_gate)
    weights, mask, _acc, idx = _build(slot0, slot1, p0, p1)
    batches = _make_gather()(xpad, idx.reshape(SLOTS))
    return (weights.reshape(NUM_TOKENS, N_EXPERTS, CAP),
            mask.reshape(NUM_TOKENS, N_EXPERTS, CAP),
            batches.reshape(N_EXPERTS, CAP, D_MODEL),
            logits.reshape(b, s, N_EXPERTS))
